# baseline (speedup 1.0000x reference)
import functools

import jax
import jax.numpy as jnp
from jax import lax
from jax.experimental import pallas as pl
from jax.experimental.pallas import tpu as pltpu
from jax.experimental.pallas import tpu_sc as plsc

_N = 10000
_E = 320000
_D = 128

_NC = 2
_NS = 16
_NW = _NC * _NS
_CH = 128
_EPW = _E // _NW
_NCHT = -(-_EPW // _CH)
_PADE = _NCHT * _CH - _EPW
_NPAD = _N + 8
_RPT = 624
_RREM = _N - _RPT * _NS

_mesh = plsc.VectorSubcoreMesh(core_axis_name="c", subcore_axis_name="s")


@functools.partial(
    pl.kernel,
    mesh=_mesh,
    out_type=jax.ShapeDtypeStruct((_NC, _N, _D), jnp.float32),
    scratch_types=[
        pltpu.VMEM((_NCHT, _CH), jnp.int32),
        pltpu.VMEM((_NCHT, _CH), jnp.int32),
        pltpu.VMEM((_CH, _D), jnp.float32),
        pltpu.VMEM_SHARED((_NPAD, _D), jnp.float32),
        pltpu.SemaphoreType.DMA,
    ],
)
def _sc_agg(x_hbm, src_hbm, dst_hbm, z_hbm, out_hbm, src_v, dst_v, buf0,
            agg_sh, sem0):
    c = lax.axis_index("c")
    s = lax.axis_index("s")
    wid = s * _NC + c
    pltpu.sync_copy(z_hbm.at[pl.ds(s * _RPT, _RPT)],
                    agg_sh.at[pl.ds(s * _RPT, _RPT)])

    @pl.when(s == 0)
    def _zero_tail():
        pltpu.sync_copy(z_hbm.at[pl.ds(_RPT * _NS, _RREM)],
                        agg_sh.at[pl.ds(_RPT * _NS, _RREM)])
    pltpu.sync_copy(src_hbm.at[wid], src_v)
    pltpu.sync_copy(dst_hbm.at[wid], dst_v)
    plsc.subcore_barrier()

    def body(j, carry):
        pltpu.async_copy(x_hbm.at[src_v.at[j]], buf0, sem0).wait()
        pltpu.sync_copy(buf0, agg_sh.at[dst_v.at[j]], add=True)
        return carry

    lax.fori_loop(0, _NCHT, body, 0)
    plsc.subcore_barrier()
    pltpu.sync_copy(agg_sh.at[pl.ds(s * _RPT, _RPT)],
                    out_hbm.at[c, pl.ds(s * _RPT, _RPT)])

    @pl.when(s == 0)
    def _write_tail():
        pltpu.sync_copy(agg_sh.at[pl.ds(_RPT * _NS, _RREM)],
                        out_hbm.at[c, pl.ds(_RPT * _NS, _RREM)])


def _mlp_body(x_ref, agg_ref, w_ref, b_ref, g_ref, be_ref, out_ref):
    h = x_ref[...] + agg_ref[0] + agg_ref[1]
    t = lax.dot_general(h, w_ref[...], (((1,), (1,)), ((), ())),
                        preferred_element_type=jnp.float32)
    t = jnp.maximum(t + b_ref[...], 0.0)
    mean = jnp.mean(t, axis=0, keepdims=True)
    ctr = t - mean
    var = jnp.mean(ctr * ctr, axis=0, keepdims=True)
    out_ref[...] = ctr * lax.rsqrt(var + 1e-5) * g_ref[...] + be_ref[...]


def _mlp(x, agg, w, b, g, be):
    return pl.pallas_call(
        _mlp_body,
        out_shape=jax.ShapeDtypeStruct((_N, _D), jnp.float32),
    )(x, agg, w, b.reshape(1, _D), g.reshape(1, _D), be.reshape(1, _D))


def kernel(x, edge_index, W1, b1, g1, be1, W2, b2, g2, be2):
    srcw = edge_index[0].astype(jnp.int32).reshape(_NW, _EPW)
    dstw = edge_index[1].astype(jnp.int32).reshape(_NW, _EPW)
    pad_src = jnp.zeros((_NW, _PADE), jnp.int32)
    pad_dst = jnp.broadcast_to(
        _N + (jnp.arange(_PADE, dtype=jnp.int32) % 8), (_NW, _PADE))
    src = jnp.concatenate([srcw, pad_src], axis=1).reshape(_NW, _NCHT, _CH)
    dst = jnp.concatenate([dstw, pad_dst], axis=1).reshape(_NW, _NCHT, _CH)
    z = jnp.zeros((_N, _D), jnp.float32)
    agg1 = _sc_agg(x, src, dst, z)
    h1 = _mlp(x, agg1, W1, b1, g1, be1)
    agg2 = _sc_agg(h1, src, dst, z)
    h2 = _mlp(h1, agg2, W2, b2, g2, be2)
    return h2


# speedup vs baseline: 1.1689x; 1.1689x over previous
import functools

import jax
import jax.numpy as jnp
from jax import lax
from jax.experimental import pallas as pl
from jax.experimental.pallas import tpu as pltpu
from jax.experimental.pallas import tpu_sc as plsc

_N = 10000
_E = 320000
_D = 128

_NC = 2
_NS = 16
_NW = _NC * _NS
_CH = 128
_EPW = _E // _NW
_NCHT = -(-_EPW // _CH)
_PADE = _NCHT * _CH - _EPW
_NPAD = _N + 8
_RPT = 624
_RREM = _N - _RPT * _NS

_mesh = plsc.VectorSubcoreMesh(core_axis_name="c", subcore_axis_name="s")


@functools.partial(
    pl.kernel,
    mesh=_mesh,
    out_type=jax.ShapeDtypeStruct((_NC, _N, _D), jnp.float32),
    scratch_types=[
        pltpu.VMEM((_NCHT, _CH), jnp.int32),
        pltpu.VMEM((_NCHT, _CH), jnp.int32),
        pltpu.VMEM((_CH, _D), jnp.float32),
        pltpu.VMEM_SHARED((_NPAD, _D), jnp.float32),
        pltpu.SemaphoreType.DMA,
    ],
)
def _sc_agg(x_hbm, src_hbm, dst_hbm, z_hbm, out_hbm, src_v, dst_v, buf0,
            agg_sh, sem0):
    c = lax.axis_index("c")
    s = lax.axis_index("s")
    wid = s * _NC + c
    pltpu.sync_copy(z_hbm.at[pl.ds(s * _RPT, _RPT)],
                    agg_sh.at[pl.ds(s * _RPT, _RPT)])

    @pl.when(s == 0)
    def _zero_tail():
        pltpu.sync_copy(z_hbm.at[pl.ds(_RPT * _NS, _RREM)],
                        agg_sh.at[pl.ds(_RPT * _NS, _RREM)])
    pltpu.sync_copy(src_hbm.at[wid], src_v)
    pltpu.sync_copy(dst_hbm.at[wid], dst_v)
    plsc.subcore_barrier()

    def body(j, carry):
        pltpu.async_copy(x_hbm.at[src_v.at[j]], buf0, sem0).wait()
        return carry

    lax.fori_loop(0, _NCHT, body, 0)
    plsc.subcore_barrier()
    pltpu.sync_copy(agg_sh.at[pl.ds(s * _RPT, _RPT)],
                    out_hbm.at[c, pl.ds(s * _RPT, _RPT)])

    @pl.when(s == 0)
    def _write_tail():
        pltpu.sync_copy(agg_sh.at[pl.ds(_RPT * _NS, _RREM)],
                        out_hbm.at[c, pl.ds(_RPT * _NS, _RREM)])


def _mlp_body(x_ref, agg_ref, w_ref, b_ref, g_ref, be_ref, out_ref):
    h = x_ref[...] + agg_ref[0] + agg_ref[1]
    t = lax.dot_general(h, w_ref[...], (((1,), (1,)), ((), ())),
                        preferred_element_type=jnp.float32)
    t = jnp.maximum(t + b_ref[...], 0.0)
    mean = jnp.mean(t, axis=0, keepdims=True)
    ctr = t - mean
    var = jnp.mean(ctr * ctr, axis=0, keepdims=True)
    out_ref[...] = ctr * lax.rsqrt(var + 1e-5) * g_ref[...] + be_ref[...]


def _mlp(x, agg, w, b, g, be):
    return pl.pallas_call(
        _mlp_body,
        out_shape=jax.ShapeDtypeStruct((_N, _D), jnp.float32),
    )(x, agg, w, b.reshape(1, _D), g.reshape(1, _D), be.reshape(1, _D))


def kernel(x, edge_index, W1, b1, g1, be1, W2, b2, g2, be2):
    srcw = edge_index[0].astype(jnp.int32).reshape(_NW, _EPW)
    dstw = edge_index[1].astype(jnp.int32).reshape(_NW, _EPW)
    pad_src = jnp.zeros((_NW, _PADE), jnp.int32)
    pad_dst = jnp.broadcast_to(
        _N + (jnp.arange(_PADE, dtype=jnp.int32) % 8), (_NW, _PADE))
    src = jnp.concatenate([srcw, pad_src], axis=1).reshape(_NW, _NCHT, _CH)
    dst = jnp.concatenate([dstw, pad_dst], axis=1).reshape(_NW, _NCHT, _CH)
    z = jnp.zeros((_N, _D), jnp.float32)
    agg1 = _sc_agg(x, src, dst, z)
    h1 = _mlp(x, agg1, W1, b1, g1, be1)
    agg2 = _sc_agg(h1, src, dst, z)
    h2 = _mlp(h1, agg2, W2, b2, g2, be2)
    return h2


# speedup vs baseline: 1.8888x; 1.6158x over previous
import functools

import jax
import jax.numpy as jnp
from jax import lax
from jax.experimental import pallas as pl
from jax.experimental.pallas import tpu as pltpu
from jax.experimental.pallas import tpu_sc as plsc

_N = 10000
_E = 320000
_D = 128

_NC = 2
_NS = 16
_NW = _NC * _NS
_CH = 80
_EPW = _E // _NW
_NCHT = _EPW // _CH
_NPAD = _N
_RPT = 624
_RREM = _N - _RPT * _NS

_mesh = plsc.VectorSubcoreMesh(core_axis_name="c", subcore_axis_name="s")


@functools.partial(
    pl.kernel,
    mesh=_mesh,
    out_type=jax.ShapeDtypeStruct((_NC, _N, _D), jnp.float32),
    scratch_types=[
        pltpu.VMEM((_NCHT, _CH), jnp.int32),
        pltpu.VMEM((8, _CH), jnp.int32),
        pltpu.VMEM((8, _CH), jnp.int32),
        pltpu.VMEM((_CH, _D), jnp.float32),
        pltpu.VMEM((_CH, _D), jnp.float32),
        pltpu.VMEM_SHARED((_NPAD, _D), jnp.float32),
        pltpu.SemaphoreType.DMA,
        pltpu.SemaphoreType.DMA,
        pltpu.SemaphoreType.DMA,
    ],
)
def _sc_agg(x_hbm, src_hbm, dst_hbm, z_hbm, out_hbm, dst_v, sia, sib, buf0,
            buf1, agg_sh, sem0, sem1, semib):
    c = lax.axis_index("c")
    s = lax.axis_index("s")
    wid = s * _NC + c
    pltpu.sync_copy(z_hbm.at[pl.ds(s * _RPT, _RPT)],
                    agg_sh.at[pl.ds(s * _RPT, _RPT)])

    @pl.when(s == 0)
    def _zero_tail():
        pltpu.sync_copy(z_hbm.at[pl.ds(_RPT * _NS, _RREM)],
                        agg_sh.at[pl.ds(_RPT * _NS, _RREM)])
    pltpu.sync_copy(dst_hbm.at[wid], dst_v)
    plsc.subcore_barrier()

    pltpu.sync_copy(src_hbm.at[wid, pl.ds(0, 1)], sia.at[pl.ds(0, 1)])
    pltpu.async_copy(x_hbm.at[sia.at[0]], buf0, sem0)
    pltpu.async_copy(src_hbm.at[wid, pl.ds(1, 1)], sib.at[pl.ds(0, 1)], semib)

    def body(i, carry):
        c0 = 2 * i
        pltpu.make_async_copy(src_hbm.at[wid, pl.ds(0, 1)],
                              sib.at[pl.ds(0, 1)], semib).wait()
        pltpu.async_copy(x_hbm.at[sib.at[0]], buf1, sem1)
        pltpu.make_async_copy(x_hbm.at[pl.ds(0, _CH)], buf0, sem0).wait()
        pltpu.sync_copy(buf0, agg_sh.at[dst_v.at[c0]], add=True)
        pltpu.sync_copy(src_hbm.at[wid, pl.ds(c0 + 2, 1)], sia.at[pl.ds(0, 1)])
        pltpu.async_copy(x_hbm.at[sia.at[0]], buf0, sem0)
        pltpu.make_async_copy(x_hbm.at[pl.ds(0, _CH)], buf1, sem1).wait()
        pltpu.sync_copy(buf1, agg_sh.at[dst_v.at[c0 + 1]], add=True)
        nxt = jnp.minimum(c0 + 3, _NCHT - 1)
        pltpu.async_copy(src_hbm.at[wid, pl.ds(nxt, 1)],
                         sib.at[pl.ds(0, 1)], semib)
        return carry

    lax.fori_loop(0, (_NCHT - 1) // 2, body, 0)
    pltpu.make_async_copy(src_hbm.at[wid, pl.ds(0, 1)],
                          sib.at[pl.ds(0, 1)], semib).wait()
    pltpu.make_async_copy(x_hbm.at[pl.ds(0, _CH)], buf0, sem0).wait()
    pltpu.sync_copy(buf0, agg_sh.at[dst_v.at[_NCHT - 1]], add=True)
    plsc.subcore_barrier()
    pltpu.sync_copy(agg_sh.at[pl.ds(s * _RPT, _RPT)],
                    out_hbm.at[c, pl.ds(s * _RPT, _RPT)])

    @pl.when(s == 0)
    def _write_tail():
        pltpu.sync_copy(agg_sh.at[pl.ds(_RPT * _NS, _RREM)],
                        out_hbm.at[c, pl.ds(_RPT * _NS, _RREM)])


def _mlp_body(x_ref, agg_ref, w_ref, b_ref, g_ref, be_ref, out_ref):
    h = x_ref[...] + agg_ref[0] + agg_ref[1]
    t = lax.dot_general(h, w_ref[...], (((1,), (1,)), ((), ())),
                        preferred_element_type=jnp.float32)
    t = jnp.maximum(t + b_ref[...], 0.0)
    mean = jnp.mean(t, axis=0, keepdims=True)
    ctr = t - mean
    var = jnp.mean(ctr * ctr, axis=0, keepdims=True)
    out_ref[...] = ctr * lax.rsqrt(var + 1e-5) * g_ref[...] + be_ref[...]


def _mlp(x, agg, w, b, g, be):
    return pl.pallas_call(
        _mlp_body,
        out_shape=jax.ShapeDtypeStruct((_N, _D), jnp.float32),
    )(x, agg, w, b.reshape(1, _D), g.reshape(1, _D), be.reshape(1, _D))


def kernel(x, edge_index, W1, b1, g1, be1, W2, b2, g2, be2):
    src = edge_index[0].astype(jnp.int32).reshape(_NW, _NCHT, _CH)
    dst = edge_index[1].astype(jnp.int32).reshape(_NW, _NCHT, _CH)
    z = jnp.zeros((_N, _D), jnp.float32)
    agg1 = _sc_agg(x, src, dst, z)
    h1 = _mlp(x, agg1, W1, b1, g1, be1)
    agg2 = _sc_agg(h1, src, dst, z)
    h2 = _mlp(h1, agg2, W2, b2, g2, be2)
    return h2


# speedup vs baseline: 2.2012x; 1.1654x over previous
import functools

import jax
import jax.numpy as jnp
from jax import lax
from jax.experimental import pallas as pl
from jax.experimental.pallas import tpu as pltpu
from jax.experimental.pallas import tpu_sc as plsc

_N = 10000
_E = 320000
_D = 128

_NC = 2
_NS = 16
_NW = _NC * _NS
_CH = 80
_EPW = _E // _NW
_NCHT = _EPW // _CH
_NPAD = _N
_RPT = 624
_RREM = _N - _RPT * _NS

_mesh = plsc.VectorSubcoreMesh(core_axis_name="c", subcore_axis_name="s")


@functools.partial(
    pl.kernel,
    mesh=_mesh,
    out_type=jax.ShapeDtypeStruct((_NC, _N, _D), jnp.float32),
    scratch_types=[
        pltpu.VMEM((_NCHT, _CH), jnp.int32),
        pltpu.VMEM((8, _CH), jnp.int32),
        pltpu.VMEM((8, _CH), jnp.int32),
        pltpu.VMEM((_CH, _D), jnp.float32),
        pltpu.VMEM((_CH, _D), jnp.float32),
        pltpu.VMEM_SHARED((_NPAD, _D), jnp.float32),
        pltpu.SemaphoreType.DMA,
        pltpu.SemaphoreType.DMA,
        pltpu.SemaphoreType.DMA,
    ],
)
def _sc_agg(x_hbm, src_hbm, dst_hbm, z_hbm, out_hbm, dst_v, sia, sib, buf0,
            buf1, agg_sh, sem0, sem1, semib):
    c = lax.axis_index("c")
    s = lax.axis_index("s")
    wid = s * _NC + c
    pltpu.sync_copy(z_hbm.at[pl.ds(s * _RPT, _RPT)],
                    agg_sh.at[pl.ds(s * _RPT, _RPT)])

    @pl.when(s == 0)
    def _zero_tail():
        pltpu.sync_copy(z_hbm.at[pl.ds(_RPT * _NS, _RREM)],
                        agg_sh.at[pl.ds(_RPT * _NS, _RREM)])
    pltpu.sync_copy(dst_hbm.at[wid], dst_v)
    plsc.subcore_barrier()

    pltpu.sync_copy(src_hbm.at[wid, pl.ds(0, 1)], sia.at[pl.ds(0, 1)])
    pltpu.async_copy(x_hbm.at[sia.at[0]], buf0, sem0)
    pltpu.async_copy(src_hbm.at[wid, pl.ds(1, 1)], sib.at[pl.ds(0, 1)], semib)

    def body(i, carry):
        c0 = 2 * i
        pltpu.make_async_copy(src_hbm.at[wid, pl.ds(0, 1)],
                              sib.at[pl.ds(0, 1)], semib).wait()
        pltpu.async_copy(x_hbm.at[sib.at[0]], buf1, sem1)
        pltpu.make_async_copy(x_hbm.at[pl.ds(0, _CH)], buf0, sem0).wait()
        pltpu.sync_copy(src_hbm.at[wid, pl.ds(c0 + 2, 1)], sia.at[pl.ds(0, 1)])
        pltpu.async_copy(x_hbm.at[sia.at[0]], buf0, sem0)
        pltpu.make_async_copy(x_hbm.at[pl.ds(0, _CH)], buf1, sem1).wait()
        nxt = jnp.minimum(c0 + 3, _NCHT - 1)
        pltpu.async_copy(src_hbm.at[wid, pl.ds(nxt, 1)],
                         sib.at[pl.ds(0, 1)], semib)
        return carry

    lax.fori_loop(0, (_NCHT - 1) // 2, body, 0)
    pltpu.make_async_copy(src_hbm.at[wid, pl.ds(0, 1)],
                          sib.at[pl.ds(0, 1)], semib).wait()
    pltpu.make_async_copy(x_hbm.at[pl.ds(0, _CH)], buf0, sem0).wait()
    pltpu.sync_copy(buf0, agg_sh.at[dst_v.at[_NCHT - 1]], add=True)
    plsc.subcore_barrier()
    pltpu.sync_copy(agg_sh.at[pl.ds(s * _RPT, _RPT)],
                    out_hbm.at[c, pl.ds(s * _RPT, _RPT)])

    @pl.when(s == 0)
    def _write_tail():
        pltpu.sync_copy(agg_sh.at[pl.ds(_RPT * _NS, _RREM)],
                        out_hbm.at[c, pl.ds(_RPT * _NS, _RREM)])


def _mlp_body(x_ref, agg_ref, w_ref, b_ref, g_ref, be_ref, out_ref):
    h = x_ref[...] + agg_ref[0] + agg_ref[1]
    t = lax.dot_general(h, w_ref[...], (((1,), (1,)), ((), ())),
                        preferred_element_type=jnp.float32)
    t = jnp.maximum(t + b_ref[...], 0.0)
    mean = jnp.mean(t, axis=0, keepdims=True)
    ctr = t - mean
    var = jnp.mean(ctr * ctr, axis=0, keepdims=True)
    out_ref[...] = ctr * lax.rsqrt(var + 1e-5) * g_ref[...] + be_ref[...]


def _mlp(x, agg, w, b, g, be):
    return pl.pallas_call(
        _mlp_body,
        out_shape=jax.ShapeDtypeStruct((_N, _D), jnp.float32),
    )(x, agg, w, b.reshape(1, _D), g.reshape(1, _D), be.reshape(1, _D))


def kernel(x, edge_index, W1, b1, g1, be1, W2, b2, g2, be2):
    src = edge_index[0].astype(jnp.int32).reshape(_NW, _NCHT, _CH)
    dst = edge_index[1].astype(jnp.int32).reshape(_NW, _NCHT, _CH)
    z = jnp.zeros((_N, _D), jnp.float32)
    agg1 = _sc_agg(x, src, dst, z)
    h1 = _mlp(x, agg1, W1, b1, g1, be1)
    agg2 = _sc_agg(h1, src, dst, z)
    h2 = _mlp(h1, agg2, W2, b2, g2, be2)
    return h2


# speedup vs baseline: 2.5689x; 1.1670x over previous
import functools

import jax
import jax.numpy as jnp
from jax import lax
from jax.experimental import pallas as pl
from jax.experimental.pallas import tpu as pltpu
from jax.experimental.pallas import tpu_sc as plsc

_N = 10000
_E = 320000
_D = 128

_NC = 2
_NS = 16
_NW = _NC * _NS
_CH = 80
_EPW = _E // _NW
_NCHT = _EPW // _CH
_NPAD = _N
_RPT = 624
_RREM = _N - _RPT * _NS

_mesh = plsc.VectorSubcoreMesh(core_axis_name="c", subcore_axis_name="s")


@functools.partial(
    pl.kernel,
    mesh=_mesh,
    out_type=jax.ShapeDtypeStruct((_NC, _N, _D), jnp.float32),
    scratch_types=[
        pltpu.VMEM((_NCHT, _CH), jnp.int32),
        pltpu.VMEM((8, _CH), jnp.int32),
        pltpu.VMEM((8, _CH), jnp.int32),
        pltpu.VMEM((8, _CH), jnp.int32),
        pltpu.VMEM((_CH, _D), jnp.float32),
        pltpu.VMEM((_CH, _D), jnp.float32),
        pltpu.VMEM((_CH, _D), jnp.float32),
        pltpu.VMEM_SHARED((_NPAD, _D), jnp.float32),
        pltpu.SemaphoreType.DMA,
        pltpu.SemaphoreType.DMA,
        pltpu.SemaphoreType.DMA,
        pltpu.SemaphoreType.DMA,
        pltpu.SemaphoreType.DMA,
        pltpu.SemaphoreType.DMA,
    ],
)
def _sc_agg(x_hbm, src_hbm, dst_hbm, z_hbm, out_hbm, dst_v, sia, sib, sic,
            buf0, buf1, buf2, agg_sh, sem0, sem1, sem2, sema, semb, semc):
    c = lax.axis_index("c")
    s = lax.axis_index("s")
    wid = s * _NC + c
    pltpu.sync_copy(z_hbm.at[pl.ds(s * _RPT, _RPT)],
                    agg_sh.at[pl.ds(s * _RPT, _RPT)])

    @pl.when(s == 0)
    def _zero_tail():
        pltpu.sync_copy(z_hbm.at[pl.ds(_RPT * _NS, _RREM)],
                        agg_sh.at[pl.ds(_RPT * _NS, _RREM)])
    pltpu.sync_copy(dst_hbm.at[wid], dst_v)
    plsc.subcore_barrier()

    def _idx_wait(sx, semx):
        pltpu.make_async_copy(src_hbm.at[wid, pl.ds(0, 1)],
                              sx.at[pl.ds(0, 1)], semx).wait()

    def _idx_fetch(j, sx, semx):
        pltpu.async_copy(src_hbm.at[wid, pl.ds(j, 1)],
                         sx.at[pl.ds(0, 1)], semx)

    def _gwait(bufn, semn):
        pltpu.make_async_copy(x_hbm.at[pl.ds(0, _CH)], bufn, semn).wait()

    pltpu.sync_copy(src_hbm.at[wid, pl.ds(0, 1)], sia.at[pl.ds(0, 1)])
    pltpu.async_copy(x_hbm.at[sia.at[0]], buf0, sem0)
    pltpu.sync_copy(src_hbm.at[wid, pl.ds(1, 1)], sib.at[pl.ds(0, 1)])
    pltpu.async_copy(x_hbm.at[sib.at[0]], buf1, sem1)
    _idx_fetch(2, sic, semc)

    def body(i, carry):
        c0 = 3 * i
        _idx_wait(sic, semc)
        pltpu.async_copy(x_hbm.at[sic.at[0]], buf2, sem2)
        _gwait(buf0, sem0)
        _idx_fetch(jnp.minimum(c0 + 3, _NCHT - 1), sia, sema)
        pltpu.sync_copy(buf0, agg_sh.at[dst_v.at[c0]], add=True)
        _idx_wait(sia, sema)
        pltpu.async_copy(x_hbm.at[sia.at[0]], buf0, sem0)
        _gwait(buf1, sem1)
        _idx_fetch(jnp.minimum(c0 + 4, _NCHT - 1), sib, semb)
        pltpu.sync_copy(buf1, agg_sh.at[dst_v.at[c0 + 1]], add=True)
        _idx_wait(sib, semb)
        pltpu.async_copy(x_hbm.at[sib.at[0]], buf1, sem1)
        _gwait(buf2, sem2)
        _idx_fetch(jnp.minimum(c0 + 5, _NCHT - 1), sic, semc)
        pltpu.sync_copy(buf2, agg_sh.at[dst_v.at[c0 + 2]], add=True)
        return carry

    lax.fori_loop(0, (_NCHT - 2) // 3, body, 0)
    _idx_wait(sic, semc)
    _gwait(buf0, sem0)
    pltpu.sync_copy(buf0, agg_sh.at[dst_v.at[_NCHT - 2]], add=True)
    _gwait(buf1, sem1)
    pltpu.sync_copy(buf1, agg_sh.at[dst_v.at[_NCHT - 1]], add=True)
    plsc.subcore_barrier()
    pltpu.sync_copy(agg_sh.at[pl.ds(s * _RPT, _RPT)],
                    out_hbm.at[c, pl.ds(s * _RPT, _RPT)])

    @pl.when(s == 0)
    def _write_tail():
        pltpu.sync_copy(agg_sh.at[pl.ds(_RPT * _NS, _RREM)],
                        out_hbm.at[c, pl.ds(_RPT * _NS, _RREM)])


def _mlp_body(x_ref, agg_ref, w_ref, b_ref, g_ref, be_ref, out_ref):
    h = x_ref[...] + agg_ref[0] + agg_ref[1]
    t = lax.dot_general(h, w_ref[...], (((1,), (1,)), ((), ())),
                        preferred_element_type=jnp.float32)
    t = jnp.maximum(t + b_ref[...], 0.0)
    mean = jnp.mean(t, axis=0, keepdims=True)
    ctr = t - mean
    var = jnp.mean(ctr * ctr, axis=0, keepdims=True)
    out_ref[...] = ctr * lax.rsqrt(var + 1e-5) * g_ref[...] + be_ref[...]


def _mlp(x, agg, w, b, g, be):
    return pl.pallas_call(
        _mlp_body,
        out_shape=jax.ShapeDtypeStruct((_N, _D), jnp.float32),
    )(x, agg, w, b.reshape(1, _D), g.reshape(1, _D), be.reshape(1, _D))


def kernel(x, edge_index, W1, b1, g1, be1, W2, b2, g2, be2):
    src = edge_index[0].astype(jnp.int32).reshape(_NW, _NCHT, _CH)
    dst = edge_index[1].astype(jnp.int32).reshape(_NW, _NCHT, _CH)
    z = jnp.zeros((_N, _D), jnp.float32)
    agg1 = _sc_agg(x, src, dst, z)
    h1 = _mlp(x, agg1, W1, b1, g1, be1)
    agg2 = _sc_agg(h1, src, dst, z)
    h2 = _mlp(h1, agg2, W2, b2, g2, be2)
    return h2


# speedup vs baseline: 2.6622x; 1.0363x over previous
import functools

import jax
import jax.numpy as jnp
from jax import lax
from jax.experimental import pallas as pl
from jax.experimental.pallas import tpu as pltpu
from jax.experimental.pallas import tpu_sc as plsc

_N = 10000
_E = 320000
_D = 128

_NC = 2
_NS = 16
_NW = _NC * _NS
_CH = 80
_EPW = _E // _NW
_NCHT = _EPW // _CH
_NPAD = _N
_RPT = 624
_RREM = _N - _RPT * _NS

_mesh = plsc.VectorSubcoreMesh(core_axis_name="c", subcore_axis_name="s")


@functools.partial(
    pl.kernel,
    mesh=_mesh,
    out_type=jax.ShapeDtypeStruct((_NC, _N, _D), jnp.float32),
    scratch_types=[
        pltpu.VMEM((8, _CH), jnp.int32),
        pltpu.VMEM((8, _CH), jnp.int32),
        pltpu.VMEM((8, _CH), jnp.int32),
        pltpu.VMEM((8, _CH), jnp.int32),
        pltpu.VMEM((8, _CH), jnp.int32),
        pltpu.VMEM((8, _CH), jnp.int32),
        pltpu.VMEM((8, _CH), jnp.int32),
        pltpu.VMEM((8, _CH), jnp.int32),
        pltpu.VMEM((_CH, _D), jnp.float32),
        pltpu.VMEM((_CH, _D), jnp.float32),
        pltpu.VMEM((_CH, _D), jnp.float32),
        pltpu.VMEM((_CH, _D), jnp.float32),
        pltpu.VMEM_SHARED((_NPAD, _D), jnp.float32),
        pltpu.SemaphoreType.DMA,
        pltpu.SemaphoreType.DMA,
        pltpu.SemaphoreType.DMA,
        pltpu.SemaphoreType.DMA,
        pltpu.SemaphoreType.DMA,
        pltpu.SemaphoreType.DMA,
        pltpu.SemaphoreType.DMA,
        pltpu.SemaphoreType.DMA,
        pltpu.SemaphoreType.DMA,
        pltpu.SemaphoreType.DMA,
        pltpu.SemaphoreType.DMA,
        pltpu.SemaphoreType.DMA,
    ],
)
def _sc_agg(x_hbm, src_hbm, dst_hbm, z_hbm, out_hbm,
            si0, si1, si2, si3, di0, di1, di2, di3, b0, b1, b2, b3,
            agg_sh, sg0, sg1, sg2, sg3, ss0, ss1, ss2, ss3,
            sd0, sd1, sd2, sd3):
    c = lax.axis_index("c")
    s = lax.axis_index("s")
    wid = s * _NC + c
    pltpu.sync_copy(z_hbm.at[pl.ds(s * _RPT, _RPT)],
                    agg_sh.at[pl.ds(s * _RPT, _RPT)])

    @pl.when(s == 0)
    def _zero_tail():
        pltpu.sync_copy(z_hbm.at[pl.ds(_RPT * _NS, _RREM)],
                        agg_sh.at[pl.ds(_RPT * _NS, _RREM)])
    plsc.subcore_barrier()

    sis = (si0, si1, si2, si3)
    dis = (di0, di1, di2, di3)
    bufs = (b0, b1, b2, b3)
    sgs = (sg0, sg1, sg2, sg3)
    sss = (ss0, ss1, ss2, ss3)
    sds = (sd0, sd1, sd2, sd3)

    def _idx_fetch(hbm, j, dbuf, semx):
        pltpu.async_copy(hbm.at[wid, pl.ds(j, 1)], dbuf.at[pl.ds(0, 1)], semx)

    def _idx_wait(hbm, dbuf, semx):
        pltpu.make_async_copy(hbm.at[wid, pl.ds(0, 1)],
                              dbuf.at[pl.ds(0, 1)], semx).wait()

    def _gwait(bufn, semn):
        pltpu.make_async_copy(x_hbm.at[pl.ds(0, _CH)], bufn, semn).wait()

    for k in range(4):
        pltpu.sync_copy(src_hbm.at[wid, pl.ds(k, 1)], sis[k].at[pl.ds(0, 1)])
        pltpu.async_copy(x_hbm.at[sis[k].at[0]], bufs[k], sgs[k])
        _idx_fetch(dst_hbm, k, dis[k], sds[k])

    def body(i, carry):
        c0 = 4 * i
        for k in range(4):
            ck = c0 + k
            nk = jnp.minimum(ck + 4, _NCHT - 1)
            _gwait(bufs[k], sgs[k])
            _idx_fetch(src_hbm, nk, sis[k], sss[k])
            _idx_wait(dst_hbm, dis[k], sds[k])
            pltpu.sync_copy(bufs[k], agg_sh.at[dis[k].at[0]], add=True)
            _idx_fetch(dst_hbm, nk, dis[k], sds[k])
            _idx_wait(src_hbm, sis[k], sss[k])
            pltpu.async_copy(x_hbm.at[sis[k].at[0]], bufs[k], sgs[k])
        return carry

    lax.fori_loop(0, (_NCHT - 1) // 4, body, 0)
    _gwait(bufs[0], sgs[0])
    _idx_wait(dst_hbm, dis[0], sds[0])
    pltpu.sync_copy(bufs[0], agg_sh.at[dis[0].at[0]], add=True)
    for k in range(1, 4):
        _gwait(bufs[k], sgs[k])
        _idx_wait(dst_hbm, dis[k], sds[k])
    plsc.subcore_barrier()
    pltpu.sync_copy(agg_sh.at[pl.ds(s * _RPT, _RPT)],
                    out_hbm.at[c, pl.ds(s * _RPT, _RPT)])

    @pl.when(s == 0)
    def _write_tail():
        pltpu.sync_copy(agg_sh.at[pl.ds(_RPT * _NS, _RREM)],
                        out_hbm.at[c, pl.ds(_RPT * _NS, _RREM)])


def _mlp_body(x_ref, agg_ref, w_ref, b_ref, g_ref, be_ref, out_ref):
    h = x_ref[...] + agg_ref[0] + agg_ref[1]
    t = lax.dot_general(h, w_ref[...], (((1,), (1,)), ((), ())),
                        preferred_element_type=jnp.float32)
    t = jnp.maximum(t + b_ref[...], 0.0)
    mean = jnp.mean(t, axis=0, keepdims=True)
    ctr = t - mean
    var = jnp.mean(ctr * ctr, axis=0, keepdims=True)
    out_ref[...] = ctr * lax.rsqrt(var + 1e-5) * g_ref[...] + be_ref[...]


def _mlp(x, agg, w, b, g, be):
    return pl.pallas_call(
        _mlp_body,
        out_shape=jax.ShapeDtypeStruct((_N, _D), jnp.float32),
    )(x, agg, w, b.reshape(1, _D), g.reshape(1, _D), be.reshape(1, _D))


def kernel(x, edge_index, W1, b1, g1, be1, W2, b2, g2, be2):
    src = edge_index[0].astype(jnp.int32).reshape(_NW, _NCHT, _CH)
    dst = edge_index[1].astype(jnp.int32).reshape(_NW, _NCHT, _CH)
    z = jnp.zeros((_N, _D), jnp.float32)
    agg1 = _sc_agg(x, src, dst, z)
    h1 = _mlp(x, agg1, W1, b1, g1, be1)
    agg2 = _sc_agg(h1, src, dst, z)
    h2 = _mlp(h1, agg2, W2, b2, g2, be2)
    return h2


# speedup vs baseline: 2.8222x; 1.0601x over previous
import functools

import jax
import jax.numpy as jnp
from jax import lax
from jax.experimental import pallas as pl
from jax.experimental.pallas import tpu as pltpu
from jax.experimental.pallas import tpu_sc as plsc

_N = 10000
_E = 320000
_D = 128

_NC = 2
_NS = 16
_NW = _NC * _NS
_CH = 80
_EPW = _E // _NW
_NCHT = _EPW // _CH
_NPAD = _N
_RPT = 624
_RREM = _N - _RPT * _NS

_mesh = plsc.VectorSubcoreMesh(core_axis_name="c", subcore_axis_name="s")


@functools.partial(
    pl.kernel,
    mesh=_mesh,
    out_type=jax.ShapeDtypeStruct((_NC, _N, _D), jnp.float32),
    scratch_types=[
        pltpu.VMEM((8, _CH), jnp.int32),
        pltpu.VMEM((8, _CH), jnp.int32),
        pltpu.VMEM((8, _CH), jnp.int32),
        pltpu.VMEM((8, _CH), jnp.int32),
        pltpu.VMEM((8, _CH), jnp.int32),
        pltpu.VMEM((8, _CH), jnp.int32),
        pltpu.VMEM((8, _CH), jnp.int32),
        pltpu.VMEM((8, _CH), jnp.int32),
        pltpu.VMEM((_CH, _D), jnp.float32),
        pltpu.VMEM((_CH, _D), jnp.float32),
        pltpu.VMEM((_CH, _D), jnp.float32),
        pltpu.VMEM((_CH, _D), jnp.float32),
        pltpu.VMEM_SHARED((_NPAD, _D), jnp.float32),
        pltpu.SemaphoreType.DMA,
        pltpu.SemaphoreType.DMA,
        pltpu.SemaphoreType.DMA,
        pltpu.SemaphoreType.DMA,
        pltpu.SemaphoreType.DMA,
        pltpu.SemaphoreType.DMA,
        pltpu.SemaphoreType.DMA,
        pltpu.SemaphoreType.DMA,
        pltpu.SemaphoreType.DMA,
        pltpu.SemaphoreType.DMA,
        pltpu.SemaphoreType.DMA,
        pltpu.SemaphoreType.DMA,
    ],
)
def _sc_agg(x_hbm, edge_hbm, z_hbm, out_hbm,
            si0, si1, si2, si3, di0, di1, di2, di3, b0, b1, b2, b3,
            agg_sh, sg0, sg1, sg2, sg3, ss0, ss1, ss2, ss3,
            sd0, sd1, sd2, sd3):
    c = lax.axis_index("c")
    s = lax.axis_index("s")
    wid = s * _NC + c

    def _init_rows(src_ref):
        pltpu.sync_copy(src_ref.at[pl.ds(s * _RPT, _RPT)],
                        agg_sh.at[pl.ds(s * _RPT, _RPT)])

        @pl.when(s == 0)
        def _tail():
            pltpu.sync_copy(src_ref.at[pl.ds(_RPT * _NS, _RREM)],
                            agg_sh.at[pl.ds(_RPT * _NS, _RREM)])

    @pl.when(c == 0)
    def _init_x():
        _init_rows(x_hbm)

    @pl.when(c == 1)
    def _init_z():
        _init_rows(z_hbm)
    plsc.subcore_barrier()

    sis = (si0, si1, si2, si3)
    dis = (di0, di1, di2, di3)
    bufs = (b0, b1, b2, b3)
    sgs = (sg0, sg1, sg2, sg3)
    sss = (ss0, ss1, ss2, ss3)
    sds = (sd0, sd1, sd2, sd3)

    def _idx_fetch(row, j, dbuf, semx):
        pltpu.async_copy(edge_hbm.at[row, wid, pl.ds(j, 1)],
                         dbuf.at[pl.ds(0, 1)], semx)

    def _idx_wait(row, dbuf, semx):
        pltpu.make_async_copy(edge_hbm.at[row, wid, pl.ds(0, 1)],
                              dbuf.at[pl.ds(0, 1)], semx).wait()

    def _gwait(bufn, semn):
        pltpu.make_async_copy(x_hbm.at[pl.ds(0, _CH)], bufn, semn).wait()

    for k in range(4):
        pltpu.sync_copy(edge_hbm.at[0, wid, pl.ds(k, 1)],
                        sis[k].at[pl.ds(0, 1)])
        pltpu.async_copy(x_hbm.at[sis[k].at[0]], bufs[k], sgs[k])
        _idx_fetch(1, k, dis[k], sds[k])

    def body(i, carry):
        c0 = 4 * i
        for k in range(4):
            ck = c0 + k
            nk = jnp.minimum(ck + 4, _NCHT - 1)
            _gwait(bufs[k], sgs[k])
            _idx_fetch(0, nk, sis[k], sss[k])
            _idx_wait(1, dis[k], sds[k])
            pltpu.sync_copy(bufs[k], agg_sh.at[dis[k].at[0]], add=True)
            _idx_fetch(1, nk, dis[k], sds[k])
            _idx_wait(0, sis[k], sss[k])
            pltpu.async_copy(x_hbm.at[sis[k].at[0]], bufs[k], sgs[k])
        return carry

    lax.fori_loop(0, (_NCHT - 1) // 4, body, 0)
    _gwait(bufs[0], sgs[0])
    _idx_wait(1, dis[0], sds[0])
    pltpu.sync_copy(bufs[0], agg_sh.at[dis[0].at[0]], add=True)
    for k in range(1, 4):
        _gwait(bufs[k], sgs[k])
        _idx_wait(1, dis[k], sds[k])
    plsc.subcore_barrier()
    pltpu.sync_copy(agg_sh.at[pl.ds(s * _RPT, _RPT)],
                    out_hbm.at[c, pl.ds(s * _RPT, _RPT)])

    @pl.when(s == 0)
    def _write_tail():
        pltpu.sync_copy(agg_sh.at[pl.ds(_RPT * _NS, _RREM)],
                        out_hbm.at[c, pl.ds(_RPT * _NS, _RREM)])


def _mlp_body(agg_ref, w_ref, b_ref, g_ref, be_ref, out_ref):
    h = agg_ref[0] + agg_ref[1]
    t = lax.dot_general(h, w_ref[...], (((1,), (1,)), ((), ())),
                        preferred_element_type=jnp.float32)
    t = jnp.maximum(t + b_ref[...], 0.0)
    mean = jnp.mean(t, axis=0, keepdims=True)
    ctr = t - mean
    var = jnp.mean(ctr * ctr, axis=0, keepdims=True)
    out_ref[...] = ctr * lax.rsqrt(var + 1e-5) * g_ref[...] + be_ref[...]


def _mlp(agg, w, b, g, be):
    return pl.pallas_call(
        _mlp_body,
        out_shape=jax.ShapeDtypeStruct((_N, _D), jnp.float32),
    )(agg, w, b.reshape(1, _D), g.reshape(1, _D), be.reshape(1, _D))


def kernel(x, edge_index, W1, b1, g1, be1, W2, b2, g2, be2):
    edges = edge_index.astype(jnp.int32).reshape(2, _NW, _NCHT, _CH)
    z = jnp.zeros((_N, _D), jnp.float32)
    agg1 = _sc_agg(x, edges, z)
    h1 = _mlp(agg1, W1, b1, g1, be1)
    agg2 = _sc_agg(h1, edges, z)
    h2 = _mlp(agg2, W2, b2, g2, be2)
    return h2


# speedup vs baseline: 2.8494x; 1.0096x over previous
import functools

import jax
import jax.numpy as jnp
from jax import lax
from jax.experimental import pallas as pl
from jax.experimental.pallas import tpu as pltpu
from jax.experimental.pallas import tpu_sc as plsc

_N = 10000
_E = 320000
_D = 128

_NC = 2
_NS = 16
_NW = _NC * _NS
_CH = 80
_EPW = _E // _NW
_NCHT = _EPW // _CH
_NPAD = _N
_RPT = 624
_RREM = _N - _RPT * _NS

_mesh = plsc.VectorSubcoreMesh(core_axis_name="c", subcore_axis_name="s")


@functools.partial(
    pl.kernel,
    mesh=_mesh,
    out_type=jax.ShapeDtypeStruct((_NC, _N, _D), jnp.float32),
    scratch_types=[
        pltpu.VMEM((8, _CH), jnp.int32),
        pltpu.VMEM((8, _CH), jnp.int32),
        pltpu.VMEM((8, _CH), jnp.int32),
        pltpu.VMEM((8, _CH), jnp.int32),
        pltpu.VMEM((8, _CH), jnp.int32),
        pltpu.VMEM((8, _CH), jnp.int32),
        pltpu.VMEM((8, _CH), jnp.int32),
        pltpu.VMEM((8, _CH), jnp.int32),
        pltpu.VMEM((_CH, _D), jnp.float32),
        pltpu.VMEM((_CH, _D), jnp.float32),
        pltpu.VMEM((_CH, _D), jnp.float32),
        pltpu.VMEM((_CH, _D), jnp.float32),
        pltpu.VMEM_SHARED((_NPAD, _D), jnp.float32),
        pltpu.SemaphoreType.DMA,
        pltpu.SemaphoreType.DMA,
        pltpu.SemaphoreType.DMA,
        pltpu.SemaphoreType.DMA,
        pltpu.SemaphoreType.DMA,
        pltpu.SemaphoreType.DMA,
        pltpu.SemaphoreType.DMA,
        pltpu.SemaphoreType.DMA,
        pltpu.SemaphoreType.DMA,
        pltpu.SemaphoreType.DMA,
        pltpu.SemaphoreType.DMA,
        pltpu.SemaphoreType.DMA,
    ],
)
def _sc_agg(x_hbm, edge_hbm, z_hbm, out_hbm,
            si0, si1, si2, si3, di0, di1, di2, di3, b0, b1, b2, b3,
            agg_sh, sg0, sg1, sg2, sg3, ss0, ss1, ss2, ss3,
            sd0, sd1, sd2, sd3):
    c = lax.axis_index("c")
    s = lax.axis_index("s")
    wid = s * _NC + c

    def _init_rows(src_ref):
        pltpu.sync_copy(src_ref.at[pl.ds(s * _RPT, _RPT)],
                        agg_sh.at[pl.ds(s * _RPT, _RPT)])

        @pl.when(s == 0)
        def _tail():
            pltpu.sync_copy(src_ref.at[pl.ds(_RPT * _NS, _RREM)],
                            agg_sh.at[pl.ds(_RPT * _NS, _RREM)])


    sis = (si0, si1, si2, si3)
    dis = (di0, di1, di2, di3)
    bufs = (b0, b1, b2, b3)
    sgs = (sg0, sg1, sg2, sg3)
    sss = (ss0, ss1, ss2, ss3)
    sds = (sd0, sd1, sd2, sd3)

    def _idx_fetch(row, j, dbuf, semx):
        pltpu.async_copy(edge_hbm.at[row, wid, pl.ds(j, 1)],
                         dbuf.at[pl.ds(0, 1)], semx)

    def _idx_wait(row, dbuf, semx):
        pltpu.make_async_copy(edge_hbm.at[row, wid, pl.ds(0, 1)],
                              dbuf.at[pl.ds(0, 1)], semx).wait()

    def _gwait(bufn, semn):
        pltpu.make_async_copy(x_hbm.at[pl.ds(0, _CH)], bufn, semn).wait()

    for k in range(4):
        _idx_fetch(0, k, sis[k], sss[k])
        _idx_fetch(1, k, dis[k], sds[k])
    for k in range(4):
        _idx_wait(0, sis[k], sss[k])
        pltpu.async_copy(x_hbm.at[sis[k].at[0]], bufs[k], sgs[k])

    @pl.when(c == 0)
    def _init_x():
        _init_rows(x_hbm)

    @pl.when(c == 1)
    def _init_z():
        _init_rows(z_hbm)
    plsc.subcore_barrier()

    def body(i, carry):
        c0 = 4 * i
        for k in range(4):
            ck = c0 + k
            nk = jnp.minimum(ck + 4, _NCHT - 1)
            _gwait(bufs[k], sgs[k])
            _idx_fetch(0, nk, sis[k], sss[k])
            _idx_wait(1, dis[k], sds[k])
            pltpu.sync_copy(bufs[k], agg_sh.at[dis[k].at[0]], add=True)
            _idx_fetch(1, nk, dis[k], sds[k])
            _idx_wait(0, sis[k], sss[k])
            pltpu.async_copy(x_hbm.at[sis[k].at[0]], bufs[k], sgs[k])
        return carry

    lax.fori_loop(0, (_NCHT - 1) // 4, body, 0)
    _gwait(bufs[0], sgs[0])
    _idx_wait(1, dis[0], sds[0])
    pltpu.sync_copy(bufs[0], agg_sh.at[dis[0].at[0]], add=True)
    for k in range(1, 4):
        _gwait(bufs[k], sgs[k])
        _idx_wait(1, dis[k], sds[k])
    plsc.subcore_barrier()
    pltpu.sync_copy(agg_sh.at[pl.ds(s * _RPT, _RPT)],
                    out_hbm.at[c, pl.ds(s * _RPT, _RPT)])

    @pl.when(s == 0)
    def _write_tail():
        pltpu.sync_copy(agg_sh.at[pl.ds(_RPT * _NS, _RREM)],
                        out_hbm.at[c, pl.ds(_RPT * _NS, _RREM)])


def _mlp_body(agg_ref, w_ref, b_ref, g_ref, be_ref, out_ref):
    h = agg_ref[0] + agg_ref[1]
    t = lax.dot_general(h, w_ref[...], (((1,), (1,)), ((), ())),
                        preferred_element_type=jnp.float32)
    t = jnp.maximum(t + b_ref[...], 0.0)
    mean = jnp.mean(t, axis=0, keepdims=True)
    ctr = t - mean
    var = jnp.mean(ctr * ctr, axis=0, keepdims=True)
    out_ref[...] = ctr * lax.rsqrt(var + 1e-5) * g_ref[...] + be_ref[...]


def _mlp(agg, w, b, g, be):
    return pl.pallas_call(
        _mlp_body,
        out_shape=jax.ShapeDtypeStruct((_N, _D), jnp.float32),
    )(agg, w, b.reshape(1, _D), g.reshape(1, _D), be.reshape(1, _D))


def kernel(x, edge_index, W1, b1, g1, be1, W2, b2, g2, be2):
    edges = edge_index.astype(jnp.int32).reshape(2, _NW, _NCHT, _CH)
    z = jnp.zeros((_N, _D), jnp.float32)
    agg1 = _sc_agg(x, edges, z)
    h1 = _mlp(agg1, W1, b1, g1, be1)
    agg2 = _sc_agg(h1, edges, z)
    h2 = _mlp(agg2, W2, b2, g2, be2)
    return h2
